# baseline (speedup 1.0000x reference)
import jax
import jax.numpy as jnp
from jax.experimental import pallas as pl
from jax.experimental.pallas import tpu as pltpu

_NODE_COUNT = 100


def _agent_mlp_kernel(
    x_hbm, W1_hbm, b1_hbm, We_hbm, be_hbm, out_hbm,
    xs, W1s, b1s, Wes, bes, outs, sem,
):
    A = out_hbm.shape[0]
    src = x_hbm.reshape(A, _NODE_COUNT, x_hbm.shape[1]).at[:, 0, :]
    copies = [
        pltpu.make_async_copy(src, xs.at[pl.ds(0, A)], sem),
        pltpu.make_async_copy(W1_hbm, W1s, sem),
        pltpu.make_async_copy(b1_hbm, b1s, sem),
        pltpu.make_async_copy(We_hbm, Wes, sem),
        pltpu.make_async_copy(be_hbm, bes, sem),
    ]
    for cp in copies:
        cp.start()
    for cp in copies:
        cp.wait()
    h = jnp.dot(xs[...], W1s[...], preferred_element_type=jnp.float32)
    h = jnp.maximum(h + b1s[...], 0.0)
    out = jnp.dot(h, Wes[...], preferred_element_type=jnp.float32) + bes[...]
    outs[...] = out[:A]
    ocp = pltpu.make_async_copy(outs, out_hbm, sem)
    ocp.start()
    ocp.wait()


def kernel(x, edge_index, node_count, W1, b1, Wc, bc, We, be):
    N, D = x.shape
    H = W1.shape[1]
    Z = We.shape[1]
    A = (N + _NODE_COUNT - 1) // _NODE_COUNT
    A_pad = -(-A // 8) * 8
    return pl.pallas_call(
        _agent_mlp_kernel,
        out_shape=jax.ShapeDtypeStruct((A, Z), jnp.float32),
        in_specs=[pl.BlockSpec(memory_space=pl.ANY)] * 5,
        out_specs=pl.BlockSpec(memory_space=pl.ANY),
        scratch_shapes=[
            pltpu.VMEM((A_pad, D), jnp.float32),
            pltpu.VMEM((D, H), jnp.float32),
            pltpu.VMEM((1, H), jnp.float32),
            pltpu.VMEM((H, Z), jnp.float32),
            pltpu.VMEM((1, Z), jnp.float32),
            pltpu.VMEM((A, Z), jnp.float32),
            pltpu.SemaphoreType.DMA,
        ],
    )(x, W1, b1.reshape(1, H), We, be.reshape(1, Z))
